# baseline (speedup 1.0000x reference)
import jax
import jax.numpy as jnp
from jax import lax
from jax.experimental import pallas as pl
from jax.experimental.pallas import tpu as pltpu

_THR = 0.3
_C25 = 0.04


def _pad128(x, val):
    h = x.shape[0]
    f = jnp.full((h, 128), val, x.dtype)
    return jnp.concatenate([f, x, f], axis=1)


def _box5_cols(xp):
    w = xp.shape[1] - 256
    acc = xp[:, 126:126 + w]
    for d in (127, 128, 129, 130):
        acc = acc + xp[:, d:d + w]
    return acc


def _box11_cols(xp):
    w = xp.shape[1] - 256
    acc = xp[:, 123:123 + w]
    for d in range(124, 134):
        acc = acc + xp[:, d:d + w]
    return acc


def _band(n, width, colvalid):
    i = lax.broadcasted_iota(jnp.int32, (n, n), 0)
    k = lax.broadcasted_iota(jnp.int32, (n, n), 1)
    d = k - i
    cond = (d >= 0) & (d < width)
    if colvalid is not None:
        cond = cond & colvalid
    return jnp.where(cond, 1.0, 0.0).astype(jnp.float32)


def _rowsum(bmat, x):
    return jnp.dot(bmat.astype(jnp.bfloat16), x.astype(jnp.bfloat16),
                   preferred_element_type=jnp.float32)


_S1 = 512
_H1 = 8
_S2 = 256
_H2 = 16


def _stats_kernel(nr_t, nr_s, nr_b, r_s, out_ref):
    s = pl.program_id(1)
    he = _S1 + 2 * _H1
    base = s * _S1 - _H1
    kg = lax.broadcasted_iota(jnp.int32, (1, he), 1) + base
    b5 = _band(he, 5, (kg >= 0) & (kg < 1024))

    gi = lax.broadcasted_iota(jnp.int32, (_S1, 1024), 0) + s * _S1
    gj = lax.broadcasted_iota(jnp.int32, (_S1, 1024), 1)
    ch = jnp.minimum(gi + 2, 1023) - jnp.maximum(gi - 2, 0) + 1
    cw = jnp.minimum(gj + 2, 1023) - jnp.maximum(gj - 2, 0) + 1
    w = (ch * cw).astype(jnp.float32)
    xsum_n = (nr_s[0, 0] + nr_s[0, 1]) + nr_s[0, 2]
    xsum_r = (r_s[0, 0] + r_s[0, 1]) + r_s[0, 2]
    wsn = jnp.sum(xsum_n * w, axis=0, keepdims=True)
    wsr = jnp.sum(xsum_r * w, axis=0, keepdims=True)

    ri = lax.broadcasted_iota(jnp.int32, (he, 1024), 0)
    rowsel = (ri >= _H1 - 2) & (ri < _H1 + _S1 - 2)
    mn = None
    mx = None
    for c in range(3):
        xe = jnp.concatenate([nr_t[0, c], nr_s[0, c], nr_b[0, c]], axis=0)
        box = _box5_cols(_pad128(_rowsum(b5, xe), 0.0))
        mnc = jnp.min(jnp.where(rowsel, box, jnp.inf), axis=0, keepdims=True)
        mxc = jnp.max(jnp.where(rowsel, box, -jnp.inf), axis=0, keepdims=True)
        mn = mnc if mn is None else jnp.minimum(mn, mnc)
        mx = mxc if mx is None else jnp.maximum(mx, mxc)

    out_ref[0, 0, 0:1, :] = wsn
    out_ref[0, 0, 1:2, :] = wsr
    out_ref[0, 0, 2:3, :] = mn
    out_ref[0, 0, 3:4, :] = mx
    out_ref[0, 0, 4:8, :] = jnp.zeros((4, 1024), jnp.float32)


def _mask_kernel(params, nr_t, nr_s, nr_b, r_t, r_s, r_b, gm_ref, ngm_ref):
    s = pl.program_id(1)
    he = _S2 + 2 * _H2
    base = s * _S2 - _H2
    kg = lax.broadcasted_iota(jnp.int32, (1, he), 1) + base
    b5 = _band(he, 5, (kg >= 0) & (kg < 1024))
    b11 = _band(he, 11, None)

    factor = params[0]
    p = params[1]
    q = params[2]

    pixmax = None
    for c in range(3):
        xn = jnp.concatenate([nr_t[0, c], nr_s[0, c], nr_b[0, c]], axis=0)
        sn = _box5_cols(_pad128(_rowsum(b5, xn), 0.0))
        xr = jnp.concatenate([r_t[0, c], r_s[0, c], r_b[0, c]], axis=0)
        sr = _box5_cols(_pad128(_rowsum(b5, xr), 0.0))
        m = jnp.clip((sn * _C25) * factor, 0.0, 1.0)
        nr2 = m * p + q
        d = jnp.abs(nr2 - sr * _C25)
        pixmax = d if pixmax is None else jnp.maximum(pixmax, d)

    ri = lax.broadcasted_iota(jnp.int32, (he, 1024), 0) + base + 2
    rv2 = (ri >= 0) & (ri < 1024)
    mask = jnp.where(pixmax > _THR, 1.0, 0.0).astype(jnp.float32)
    mask = jnp.where(rv2, mask, 1.0)
    maskp = _pad128(mask, 1.0)

    ews = _box11_cols(_rowsum(b11, maskp))
    er = jnp.where(ews == 121.0, 1.0, 0.0).astype(jnp.float32)
    ri7 = lax.broadcasted_iota(jnp.int32, (he, 1024), 0) + base + 7
    er = jnp.where((ri7 >= 0) & (ri7 < 1024), er, 0.0)

    dws = _box11_cols(_rowsum(b11, _pad128(er, 0.0)))
    ghost = jnp.where(dws > 0.5, 1.0, 0.0).astype(jnp.float32)
    ghost = ghost[_H2 - 12:_H2 - 12 + _S2, :]
    nghost = 1.0 - ghost
    for c in range(3):
        gm_ref[0, c] = ghost
        ngm_ref[0, c] = nghost


def kernel(non_refer, refer):
    b, c, h, w = non_refer.shape
    f32 = jnp.float32
    n1 = _S1 // _H1
    nb1 = h // _H1 - 1

    stats = pl.pallas_call(
        _stats_kernel,
        grid=(b, h // _S1),
        in_specs=[
            pl.BlockSpec((1, c, _H1, w),
                         lambda i, s: (i, 0, jnp.clip(s * n1 - 1, 0, nb1), 0)),
            pl.BlockSpec((1, c, _S1, w), lambda i, s: (i, 0, s, 0)),
            pl.BlockSpec((1, c, _H1, w),
                         lambda i, s: (i, 0, jnp.clip((s + 1) * n1, 0, nb1), 0)),
            pl.BlockSpec((1, c, _S1, w), lambda i, s: (i, 0, s, 0)),
        ],
        out_specs=pl.BlockSpec((1, 1, 8, w), lambda i, s: (i, s, 0, 0)),
        out_shape=jax.ShapeDtypeStruct((b, h // _S1, 8, w), f32),
        compiler_params=pltpu.CompilerParams(
            dimension_semantics=("parallel", "arbitrary"),
            vmem_limit_bytes=48 * 1024 * 1024,
        ),
        name="getmask_stats",
    )(non_refer, non_refer, non_refer, refer)

    wsn = jnp.sum(stats[:, :, 0, :])
    wsr = jnp.sum(stats[:, :, 1, :])
    mn_s = jnp.min(stats[:, :, 2, :])
    mx_s = jnp.max(stats[:, :, 3, :])

    factor = wsr / wsn
    mn_b = mn_s * _C25
    mx_b = mx_s * _C25
    mn_m = jnp.clip(mn_b * factor, 0.0, 1.0)
    mx_m = jnp.clip(mx_b * factor, 0.0, 1.0)
    p = (mx_b - mn_b) / (mx_m - mn_m)
    q = mn_b - mn_m * p
    params = jnp.stack([factor, p, q]).astype(f32)

    n2 = _S2 // _H2
    nb2 = h // _H2 - 1
    big = jax.ShapeDtypeStruct((b, c, h, w), f32)
    ghost, nghost = pl.pallas_call(
        _mask_kernel,
        grid=(b, h // _S2),
        in_specs=[
            pl.BlockSpec(memory_space=pltpu.SMEM),
            pl.BlockSpec((1, c, _H2, w),
                         lambda i, s: (i, 0, jnp.clip(s * n2 - 1, 0, nb2), 0)),
            pl.BlockSpec((1, c, _S2, w), lambda i, s: (i, 0, s, 0)),
            pl.BlockSpec((1, c, _H2, w),
                         lambda i, s: (i, 0, jnp.clip((s + 1) * n2, 0, nb2), 0)),
            pl.BlockSpec((1, c, _H2, w),
                         lambda i, s: (i, 0, jnp.clip(s * n2 - 1, 0, nb2), 0)),
            pl.BlockSpec((1, c, _S2, w), lambda i, s: (i, 0, s, 0)),
            pl.BlockSpec((1, c, _H2, w),
                         lambda i, s: (i, 0, jnp.clip((s + 1) * n2, 0, nb2), 0)),
        ],
        out_specs=[
            pl.BlockSpec((1, c, _S2, w), lambda i, s: (i, 0, s, 0)),
            pl.BlockSpec((1, c, _S2, w), lambda i, s: (i, 0, s, 0)),
        ],
        out_shape=[big, big],
        compiler_params=pltpu.CompilerParams(
            dimension_semantics=("parallel", "arbitrary"),
            vmem_limit_bytes=48 * 1024 * 1024,
        ),
        name="getmask_fused",
    )(params, non_refer, non_refer, non_refer, refer, refer, refer)

    return (ghost, nghost)


# speedup vs baseline: 1.7642x; 1.7642x over previous
import jax
import jax.numpy as jnp
from jax import lax
from jax.experimental import pallas as pl
from jax.experimental.pallas import tpu as pltpu

_THR = 0.3
_C25 = 0.04


def _shift_c(x, d, fill):
    h = x.shape[0]
    f = jnp.full((h, abs(d)), fill, x.dtype)
    if d > 0:
        return jnp.concatenate([x[:, d:], f], axis=1)
    return jnp.concatenate([f, x[:, :d]], axis=1)


def _sum5_cols(x):
    s1 = (x + _shift_c(x, 1, 0.0)) + _shift_c(x, -1, 0.0)
    return (s1 + _shift_c(x, 2, 0.0)) + _shift_c(x, -2, 0.0)


def _csum11_cols(x):
    s2 = x + _shift_c(x, 1, 0.0)
    s4 = s2 + _shift_c(s2, 2, 0.0)
    s8 = s4 + _shift_c(s4, 4, 0.0)
    return (s8 + _shift_c(s2, 8, 0.0)) + _shift_c(x, 10, 0.0)


def _band(n, lo, hi, colvalid):
    i = lax.broadcasted_iota(jnp.int32, (n, n), 0)
    k = lax.broadcasted_iota(jnp.int32, (n, n), 1)
    d = k - i
    cond = (d >= lo) & (d <= hi)
    if colvalid is not None:
        cond = cond & colvalid
    return jnp.where(cond, 1.0, 0.0).astype(jnp.float32)


def _rowsum(bmat, x):
    return jnp.dot(bmat.astype(jnp.bfloat16), x.astype(jnp.bfloat16),
                   preferred_element_type=jnp.float32)


_S1 = 512
_H1 = 8
_S2 = 256
_H2 = 16


def _blur_stats_kernel(nr_t, nr_s, nr_b, r_t, r_s, r_b, out_ref, snb_ref,
                       srb_ref):
    s = pl.program_id(1)
    he = _S1 + 2 * _H1
    base = s * _S1 - _H1
    kg = lax.broadcasted_iota(jnp.int32, (1, he), 1) + base
    b5 = _band(he, _H1 - 2, _H1 + 2, (kg >= 0) & (kg < 1024))

    gi = lax.broadcasted_iota(jnp.int32, (_S1, 1024), 0) + s * _S1
    gj = lax.broadcasted_iota(jnp.int32, (_S1, 1024), 1)
    ch = jnp.minimum(gi + 2, 1023) - jnp.maximum(gi - 2, 0) + 1
    cw = jnp.minimum(gj + 2, 1023) - jnp.maximum(gj - 2, 0) + 1
    w = (ch * cw).astype(jnp.float32)
    xsum_n = (nr_s[0, 0] + nr_s[0, 1]) + nr_s[0, 2]
    xsum_r = (r_s[0, 0] + r_s[0, 1]) + r_s[0, 2]
    wsn = jnp.sum(xsum_n * w, axis=0, keepdims=True)
    wsr = jnp.sum(xsum_r * w, axis=0, keepdims=True)

    mn = None
    mx = None
    for c in range(3):
        xe = jnp.concatenate([nr_t[0, c], nr_s[0, c], nr_b[0, c]], axis=0)
        box = _sum5_cols(_rowsum(b5, xe)[0:_S1])
        boxb = box.astype(jnp.bfloat16)
        snb_ref[0, c] = boxb
        mnc = jnp.min(boxb, axis=0, keepdims=True).astype(jnp.float32)
        mxc = jnp.max(boxb, axis=0, keepdims=True).astype(jnp.float32)
        mn = mnc if mn is None else jnp.minimum(mn, mnc)
        mx = mxc if mx is None else jnp.maximum(mx, mxc)
        xr = jnp.concatenate([r_t[0, c], r_s[0, c], r_b[0, c]], axis=0)
        srb_ref[0, c] = _sum5_cols(_rowsum(b5, xr)[0:_S1]).astype(jnp.bfloat16)

    out_ref[0, 0, 0:1, :] = wsn
    out_ref[0, 0, 1:2, :] = wsr
    out_ref[0, 0, 2:3, :] = mn
    out_ref[0, 0, 3:4, :] = mx
    out_ref[0, 0, 4:8, :] = jnp.zeros((4, 1024), jnp.float32)


def _mask_kernel(params, sn_t, sn_s, sn_b, sr_t, sr_s, sr_b, gm_ref, ngm_ref):
    s = pl.program_id(1)
    he = _S2 + 2 * _H2
    base = s * _S2 - _H2
    b11 = _band(he, -5, 5, None)

    factor = params[0]
    p = params[1]
    q = params[2]

    pixmax = None
    for c in range(3):
        sn = jnp.concatenate([sn_t[0, c], sn_s[0, c], sn_b[0, c]],
                             axis=0).astype(jnp.float32)
        sr = jnp.concatenate([sr_t[0, c], sr_s[0, c], sr_b[0, c]],
                             axis=0).astype(jnp.float32)
        m = jnp.clip((sn * _C25) * factor, 0.0, 1.0)
        nr2 = m * p + q
        d = jnp.abs(nr2 - sr * _C25)
        pixmax = d if pixmax is None else jnp.maximum(pixmax, d)

    ri = lax.broadcasted_iota(jnp.int32, (he, 1024), 0) + base
    rv = (ri >= 0) & (ri < 1024)
    mask = jnp.where(pixmax > _THR, 1.0, 0.0).astype(jnp.float32)
    mask = jnp.where(rv, mask, 1.0)
    ones = jnp.ones((he, 128), jnp.float32)
    maskp = jnp.concatenate([ones, mask, ones], axis=1)

    ews = _csum11_cols(_rowsum(b11, maskp))
    er = jnp.where(ews == 121.0, 1.0, 0.0).astype(jnp.float32)
    rie = lax.broadcasted_iota(jnp.int32, (he, 1280), 0) + base
    cj = lax.broadcasted_iota(jnp.int32, (he, 1280), 1)
    okd = (rie >= 0) & (rie < 1024) & (cj >= 123) & (cj < 1147)
    er = jnp.where(okd, er, 0.0)

    dws = _csum11_cols(_rowsum(b11, er))
    ghost = jnp.where(dws > 0.5, 1.0, 0.0).astype(jnp.float32)
    ghost = ghost[_H2:_H2 + _S2, 118:118 + 1024]
    nghost = 1.0 - ghost
    for c in range(3):
        gm_ref[0, c] = ghost
        ngm_ref[0, c] = nghost


def kernel(non_refer, refer):
    b, c, h, w = non_refer.shape
    f32 = jnp.float32
    bf16 = jnp.bfloat16
    n1 = _S1 // _H1
    nb1 = h // _H1 - 1

    boxed = jax.ShapeDtypeStruct((b, c, h, w), bf16)
    stats, snb, srb = pl.pallas_call(
        _blur_stats_kernel,
        grid=(b, h // _S1),
        in_specs=[
            pl.BlockSpec((1, c, _H1, w),
                         lambda i, s: (i, 0, jnp.clip(s * n1 - 1, 0, nb1), 0)),
            pl.BlockSpec((1, c, _S1, w), lambda i, s: (i, 0, s, 0)),
            pl.BlockSpec((1, c, _H1, w),
                         lambda i, s: (i, 0, jnp.clip((s + 1) * n1, 0, nb1), 0)),
            pl.BlockSpec((1, c, _H1, w),
                         lambda i, s: (i, 0, jnp.clip(s * n1 - 1, 0, nb1), 0)),
            pl.BlockSpec((1, c, _S1, w), lambda i, s: (i, 0, s, 0)),
            pl.BlockSpec((1, c, _H1, w),
                         lambda i, s: (i, 0, jnp.clip((s + 1) * n1, 0, nb1), 0)),
        ],
        out_specs=[
            pl.BlockSpec((1, 1, 8, w), lambda i, s: (i, s, 0, 0)),
            pl.BlockSpec((1, c, _S1, w), lambda i, s: (i, 0, s, 0)),
            pl.BlockSpec((1, c, _S1, w), lambda i, s: (i, 0, s, 0)),
        ],
        out_shape=[jax.ShapeDtypeStruct((b, h // _S1, 8, w), f32),
                   boxed, boxed],
        compiler_params=pltpu.CompilerParams(
            dimension_semantics=("parallel", "arbitrary"),
            vmem_limit_bytes=52 * 1024 * 1024,
        ),
        name="getmask_blurstats",
    )(non_refer, non_refer, non_refer, refer, refer, refer)

    wsn = jnp.sum(stats[:, :, 0, :])
    wsr = jnp.sum(stats[:, :, 1, :])
    mn_s = jnp.min(stats[:, :, 2, :])
    mx_s = jnp.max(stats[:, :, 3, :])

    factor = wsr / wsn
    mn_b = mn_s * _C25
    mx_b = mx_s * _C25
    mn_m = jnp.clip(mn_b * factor, 0.0, 1.0)
    mx_m = jnp.clip(mx_b * factor, 0.0, 1.0)
    p = (mx_b - mn_b) / (mx_m - mn_m)
    q = mn_b - mn_m * p
    params = jnp.stack([factor, p, q]).astype(f32)

    n2 = _S2 // _H2
    nb2 = h // _H2 - 1
    big = jax.ShapeDtypeStruct((b, c, h, w), f32)
    ghost, nghost = pl.pallas_call(
        _mask_kernel,
        grid=(b, h // _S2),
        in_specs=[
            pl.BlockSpec(memory_space=pltpu.SMEM),
            pl.BlockSpec((1, c, _H2, w),
                         lambda i, s: (i, 0, jnp.clip(s * n2 - 1, 0, nb2), 0)),
            pl.BlockSpec((1, c, _S2, w), lambda i, s: (i, 0, s, 0)),
            pl.BlockSpec((1, c, _H2, w),
                         lambda i, s: (i, 0, jnp.clip((s + 1) * n2, 0, nb2), 0)),
            pl.BlockSpec((1, c, _H2, w),
                         lambda i, s: (i, 0, jnp.clip(s * n2 - 1, 0, nb2), 0)),
            pl.BlockSpec((1, c, _S2, w), lambda i, s: (i, 0, s, 0)),
            pl.BlockSpec((1, c, _H2, w),
                         lambda i, s: (i, 0, jnp.clip((s + 1) * n2, 0, nb2), 0)),
        ],
        out_specs=[
            pl.BlockSpec((1, c, _S2, w), lambda i, s: (i, 0, s, 0)),
            pl.BlockSpec((1, c, _S2, w), lambda i, s: (i, 0, s, 0)),
        ],
        out_shape=[big, big],
        compiler_params=pltpu.CompilerParams(
            dimension_semantics=("parallel", "arbitrary"),
            vmem_limit_bytes=48 * 1024 * 1024,
        ),
        name="getmask_mask",
    )(params, snb, snb, snb, srb, srb, srb)

    return (ghost, nghost)


# speedup vs baseline: 1.7656x; 1.0008x over previous
import jax
import jax.numpy as jnp
from jax import lax
from jax.experimental import pallas as pl
from jax.experimental.pallas import tpu as pltpu

_THR = 0.3
_C25 = 0.04


def _shift_c(x, d, fill):
    h = x.shape[0]
    f = jnp.full((h, abs(d)), fill, x.dtype)
    if d > 0:
        return jnp.concatenate([x[:, d:], f], axis=1)
    return jnp.concatenate([f, x[:, :d]], axis=1)


def _sum5_cols(x):
    s1 = (x + _shift_c(x, 1, 0.0)) + _shift_c(x, -1, 0.0)
    return (s1 + _shift_c(x, 2, 0.0)) + _shift_c(x, -2, 0.0)


def _csum11_cols(x):
    s2 = x + _shift_c(x, 1, 0.0)
    s4 = s2 + _shift_c(s2, 2, 0.0)
    s8 = s4 + _shift_c(s4, 4, 0.0)
    return (s8 + _shift_c(s2, 8, 0.0)) + _shift_c(x, 10, 0.0)


def _band(n, lo, hi, colvalid):
    i = lax.broadcasted_iota(jnp.int32, (n, n), 0)
    k = lax.broadcasted_iota(jnp.int32, (n, n), 1)
    d = k - i
    cond = (d >= lo) & (d <= hi)
    if colvalid is not None:
        cond = cond & colvalid
    return jnp.where(cond, 1.0, 0.0).astype(jnp.float32)


def _rowsum(bmat, x):
    return jnp.dot(bmat.astype(jnp.bfloat16), x.astype(jnp.bfloat16),
                   preferred_element_type=jnp.float32)


_S1 = 512
_H1 = 8
_S2 = 256
_H2 = 16


def _blur_stats_kernel(nr_t, nr_s, nr_b, r_t, r_s, r_b, out_ref, snb_ref,
                       srb_ref):
    s = pl.program_id(1)
    he = _S1 + 2 * _H1
    base = s * _S1 - _H1
    kg = lax.broadcasted_iota(jnp.int32, (1, he), 1) + base
    b5 = _band(he, _H1 - 2, _H1 + 2, (kg >= 0) & (kg < 1024))

    gi = lax.broadcasted_iota(jnp.int32, (_S1, 1024), 0) + s * _S1
    gj = lax.broadcasted_iota(jnp.int32, (_S1, 1024), 1)
    ch = jnp.minimum(gi + 2, 1023) - jnp.maximum(gi - 2, 0) + 1
    cw = jnp.minimum(gj + 2, 1023) - jnp.maximum(gj - 2, 0) + 1
    w = (ch * cw).astype(jnp.float32)
    xsum_n = (nr_s[0, 0] + nr_s[0, 1]) + nr_s[0, 2]
    xsum_r = (r_s[0, 0] + r_s[0, 1]) + r_s[0, 2]
    wsn = jnp.sum(xsum_n * w, axis=0, keepdims=True)
    wsr = jnp.sum(xsum_r * w, axis=0, keepdims=True)

    mn = None
    mx = None
    for c in range(3):
        xe = jnp.concatenate([nr_t[0, c], nr_s[0, c], nr_b[0, c]], axis=0)
        boxb = _sum5_cols(_rowsum(b5, xe)[0:_S1]).astype(jnp.bfloat16)
        snb_ref[0, c] = boxb
        mnc = jnp.min(boxb, axis=0, keepdims=True).astype(jnp.float32)
        mxc = jnp.max(boxb, axis=0, keepdims=True).astype(jnp.float32)
        mn = mnc if mn is None else jnp.minimum(mn, mnc)
        mx = mxc if mx is None else jnp.maximum(mx, mxc)
        xr = jnp.concatenate([r_t[0, c], r_s[0, c], r_b[0, c]], axis=0)
        srb_ref[0, c] = _sum5_cols(_rowsum(b5, xr)[0:_S1]).astype(jnp.bfloat16)

    out_ref[0, 0, 0:1, :] = wsn
    out_ref[0, 0, 1:2, :] = wsr
    out_ref[0, 0, 2:3, :] = mn
    out_ref[0, 0, 3:4, :] = mx
    out_ref[0, 0, 4:8, :] = jnp.zeros((4, 1024), jnp.float32)


def _mask_kernel(params, sn_t, sn_s, sn_b, sr_t, sr_s, sr_b, gm_ref, ngm_ref):
    s = pl.program_id(1)
    he = _S2 + 2 * _H2
    base = s * _S2 - _H2
    b11 = _band(he, -5, 5, None)

    factor = params[0]
    p = params[1]
    q = params[2]

    sn = jnp.concatenate(
        [jnp.concatenate([sn_t[0, c], sn_s[0, c], sn_b[0, c]], axis=0)
         for c in range(3)], axis=0).astype(jnp.float32)
    sr = jnp.concatenate(
        [jnp.concatenate([sr_t[0, c], sr_s[0, c], sr_b[0, c]], axis=0)
         for c in range(3)], axis=0).astype(jnp.float32)
    m = jnp.clip((sn * _C25) * factor, 0.0, 1.0)
    nr2 = m * p + q
    d = jnp.abs(nr2 - sr * _C25)
    pixmax = jnp.maximum(jnp.maximum(d[0:he], d[he:2 * he]), d[2 * he:3 * he])

    ri = lax.broadcasted_iota(jnp.int32, (he, 1024), 0) + base
    rv = (ri >= 0) & (ri < 1024)
    mask = jnp.where(pixmax > _THR, 1.0, 0.0).astype(jnp.float32)
    mask = jnp.where(rv, mask, 1.0)
    ones = jnp.ones((he, 128), jnp.float32)
    maskp = jnp.concatenate([ones, mask, ones], axis=1)

    ews = _csum11_cols(_rowsum(b11, maskp))
    er = jnp.where(ews == 121.0, 1.0, 0.0).astype(jnp.float32)
    rie = lax.broadcasted_iota(jnp.int32, (he, 1280), 0) + base
    cj = lax.broadcasted_iota(jnp.int32, (he, 1280), 1)
    okd = (rie >= 0) & (rie < 1024) & (cj >= 123) & (cj < 1147)
    er = jnp.where(okd, er, 0.0)

    dws = _csum11_cols(_rowsum(b11, er))
    ghost = jnp.where(dws > 0.5, 1.0, 0.0).astype(jnp.float32)
    ghost = ghost[_H2:_H2 + _S2, 118:118 + 1024]
    nghost = 1.0 - ghost
    for c in range(3):
        gm_ref[0, c] = ghost
        ngm_ref[0, c] = nghost


def kernel(non_refer, refer):
    b, c, h, w = non_refer.shape
    f32 = jnp.float32
    bf16 = jnp.bfloat16
    n1 = _S1 // _H1
    nb1 = h // _H1 - 1

    boxed = jax.ShapeDtypeStruct((b, c, h, w), bf16)
    stats, snb, srb = pl.pallas_call(
        _blur_stats_kernel,
        grid=(b, h // _S1),
        in_specs=[
            pl.BlockSpec((1, c, _H1, w),
                         lambda i, s: (i, 0, jnp.clip(s * n1 - 1, 0, nb1), 0)),
            pl.BlockSpec((1, c, _S1, w), lambda i, s: (i, 0, s, 0)),
            pl.BlockSpec((1, c, _H1, w),
                         lambda i, s: (i, 0, jnp.clip((s + 1) * n1, 0, nb1), 0)),
            pl.BlockSpec((1, c, _H1, w),
                         lambda i, s: (i, 0, jnp.clip(s * n1 - 1, 0, nb1), 0)),
            pl.BlockSpec((1, c, _S1, w), lambda i, s: (i, 0, s, 0)),
            pl.BlockSpec((1, c, _H1, w),
                         lambda i, s: (i, 0, jnp.clip((s + 1) * n1, 0, nb1), 0)),
        ],
        out_specs=[
            pl.BlockSpec((1, 1, 8, w), lambda i, s: (i, s, 0, 0)),
            pl.BlockSpec((1, c, _S1, w), lambda i, s: (i, 0, s, 0)),
            pl.BlockSpec((1, c, _S1, w), lambda i, s: (i, 0, s, 0)),
        ],
        out_shape=[jax.ShapeDtypeStruct((b, h // _S1, 8, w), f32),
                   boxed, boxed],
        compiler_params=pltpu.CompilerParams(
            dimension_semantics=("parallel", "arbitrary"),
            vmem_limit_bytes=52 * 1024 * 1024,
        ),
        name="getmask_blurstats",
    )(non_refer, non_refer, non_refer, refer, refer, refer)

    wsn = jnp.sum(stats[:, :, 0, :])
    wsr = jnp.sum(stats[:, :, 1, :])
    mn_s = jnp.min(stats[:, :, 2, :])
    mx_s = jnp.max(stats[:, :, 3, :])

    factor = wsr / wsn
    mn_b = mn_s * _C25
    mx_b = mx_s * _C25
    mn_m = jnp.clip(mn_b * factor, 0.0, 1.0)
    mx_m = jnp.clip(mx_b * factor, 0.0, 1.0)
    p = (mx_b - mn_b) / (mx_m - mn_m)
    q = mn_b - mn_m * p
    params = jnp.stack([factor, p, q]).astype(f32)

    n2 = _S2 // _H2
    nb2 = h // _H2 - 1
    big = jax.ShapeDtypeStruct((b, c, h, w), f32)
    ghost, nghost = pl.pallas_call(
        _mask_kernel,
        grid=(b, h // _S2),
        in_specs=[
            pl.BlockSpec(memory_space=pltpu.SMEM),
            pl.BlockSpec((1, c, _H2, w),
                         lambda i, s: (i, 0, jnp.clip(s * n2 - 1, 0, nb2), 0)),
            pl.BlockSpec((1, c, _S2, w), lambda i, s: (i, 0, s, 0)),
            pl.BlockSpec((1, c, _H2, w),
                         lambda i, s: (i, 0, jnp.clip((s + 1) * n2, 0, nb2), 0)),
            pl.BlockSpec((1, c, _H2, w),
                         lambda i, s: (i, 0, jnp.clip(s * n2 - 1, 0, nb2), 0)),
            pl.BlockSpec((1, c, _S2, w), lambda i, s: (i, 0, s, 0)),
            pl.BlockSpec((1, c, _H2, w),
                         lambda i, s: (i, 0, jnp.clip((s + 1) * n2, 0, nb2), 0)),
        ],
        out_specs=[
            pl.BlockSpec((1, c, _S2, w), lambda i, s: (i, 0, s, 0)),
            pl.BlockSpec((1, c, _S2, w), lambda i, s: (i, 0, s, 0)),
        ],
        out_shape=[big, big],
        compiler_params=pltpu.CompilerParams(
            dimension_semantics=("parallel", "arbitrary"),
            vmem_limit_bytes=48 * 1024 * 1024,
        ),
        name="getmask_mask",
    )(params, snb, snb, snb, srb, srb, srb)

    return (ghost, nghost)
